# baseline (speedup 1.0000x reference)
import functools

import jax
import jax.numpy as jnp
from jax import lax
from jax.experimental import pallas as pl
from jax.experimental.pallas import tpu as pltpu
from jax.experimental.pallas import tpu_sc as plsc

_B = 256

_NC = 2
_NS = 16
_NW = _NC * _NS
_CHUNK = 64


def _sc_mesh():
    return plsc.VectorSubcoreMesh(core_axis_name="c", subcore_axis_name="s",
                                  num_cores=_NC, num_subcores=_NS)


@functools.cache
def _row_gather_kernel(n_rows, dim):
    per_w = n_rows // _NW
    n_chunks = per_w // _CHUNK
    C = _CHUNK

    @functools.partial(
        pl.kernel,
        out_type=jax.ShapeDtypeStruct((n_rows, dim), jnp.float32),
        mesh=_sc_mesh(),
        scratch_types=[
            pltpu.VMEM((2, C), jnp.int32),
            pltpu.VMEM((2, C, dim), jnp.float32),
            pltpu.SemaphoreType.DMA,
            pltpu.SemaphoreType.DMA,
        ],
    )
    def gather_k(table_hbm, idx_hbm, out_hbm, idx_v, rows_v, gsem, wsem):
        wid = lax.axis_index("s") * _NC + lax.axis_index("c")
        base = wid * per_w
        g = [None] * n_chunks
        w = [None] * n_chunks
        for j in range(n_chunks):
            b = j % 2
            if j >= 2:
                w[j - 2].wait()
            pltpu.sync_copy(idx_hbm.at[pl.ds(base + j * C, C)], idx_v.at[b])
            g[j] = pltpu.async_copy(table_hbm.at[idx_v.at[b]], rows_v.at[b],
                                    gsem)
            if j >= 1:
                g[j - 1].wait()
                w[j - 1] = pltpu.async_copy(
                    rows_v.at[1 - b],
                    out_hbm.at[pl.ds(base + (j - 1) * C, C)], wsem)
        g[n_chunks - 1].wait()
        w[n_chunks - 1] = pltpu.async_copy(
            rows_v.at[(n_chunks - 1) % 2],
            out_hbm.at[pl.ds(base + (n_chunks - 1) * C, C)], wsem)
        w[n_chunks - 2].wait()
        w[n_chunks - 1].wait()

    return gather_k


@functools.cache
def _row_scatter_kernel(n_rows, dim):
    per_w = n_rows // _NW
    n_chunks = per_w // _CHUNK
    C = _CHUNK

    @functools.partial(
        pl.kernel,
        out_type=jax.ShapeDtypeStruct((n_rows, dim), jnp.float32),
        mesh=_sc_mesh(),
        scratch_types=[
            pltpu.VMEM((2, C), jnp.int32),
            pltpu.VMEM((2, C, dim), jnp.float32),
            pltpu.SemaphoreType.DMA,
            pltpu.SemaphoreType.DMA,
        ],
    )
    def scatter_k(rows_hbm, idx_hbm, out_hbm, idx_v, rows_v, rsem, ssem):
        wid = lax.axis_index("s") * _NC + lax.axis_index("c")
        base = wid * per_w
        r = [None] * n_chunks
        s = [None] * n_chunks
        for j in range(n_chunks):
            b = j % 2
            if j >= 2:
                s[j - 2].wait()
            pltpu.sync_copy(idx_hbm.at[pl.ds(base + j * C, C)], idx_v.at[b])
            r[j] = pltpu.async_copy(rows_hbm.at[pl.ds(base + j * C, C)],
                                    rows_v.at[b], rsem)
            if j >= 1:
                r[j - 1].wait()
                s[j - 1] = pltpu.async_copy(rows_v.at[1 - b],
                                            out_hbm.at[idx_v.at[1 - b]], ssem)
        r[n_chunks - 1].wait()
        s[n_chunks - 1] = pltpu.async_copy(
            rows_v.at[(n_chunks - 1) % 2],
            out_hbm.at[idx_v.at[(n_chunks - 1) % 2]], ssem)
        s[n_chunks - 2].wait()
        s[n_chunks - 1].wait()

    return scatter_k


def _mlp_step(sg_ref, st_ref, off_ref, x_ref, w1_ref, b1_ref, w2_ref, b2_ref,
              o_ref):
    s = pl.program_id(0)
    g = sg_ref[s]
    t = st_ref[s]
    lo = off_ref[g]
    hi = off_ref[g + 1]
    x = x_ref[...]
    h = lax.dot_general(x, w1_ref[0], (((1,), (1,)), ((), ())),
                        preferred_element_type=jnp.float32)
    h = h + b1_ref[0]
    h = 0.5 * h * (1.0 + lax.erf(h * (2.0 ** -0.5)))
    o = lax.dot_general(h, w2_ref[0], (((1,), (1,)), ((), ())),
                        preferred_element_type=jnp.float32)
    o = o + b2_ref[0]
    row = t * _B + lax.broadcasted_iota(jnp.int32, (_B, 1), 0)
    mask = (row >= lo) & (row < hi)
    o_ref[...] = jnp.where(mask, o, o_ref[...])


def _grouped_mlp(x_sorted, offsets, step_g, step_t, W1, b1, W2, b2, *,
                 interpret=False):
    N, D_IN = x_sorted.shape
    Lp, D_HID, _ = W1.shape
    D_OUT = W2.shape[1]
    S = step_g.shape[0]
    grid_spec = pltpu.PrefetchScalarGridSpec(
        num_scalar_prefetch=3,
        grid=(S,),
        in_specs=[
            pl.BlockSpec((_B, D_IN), lambda s, sg, st, off: (st[s], 0)),
            pl.BlockSpec((1, D_HID, D_IN), lambda s, sg, st, off: (sg[s], 0, 0)),
            pl.BlockSpec((1, 1, D_HID), lambda s, sg, st, off: (sg[s], 0, 0)),
            pl.BlockSpec((1, D_OUT, D_HID), lambda s, sg, st, off: (sg[s], 0, 0)),
            pl.BlockSpec((1, 1, D_OUT), lambda s, sg, st, off: (sg[s], 0, 0)),
        ],
        out_specs=pl.BlockSpec((_B, D_OUT), lambda s, sg, st, off: (st[s], 0)),
    )
    return pl.pallas_call(
        _mlp_step,
        grid_spec=grid_spec,
        out_shape=jax.ShapeDtypeStruct((N, D_OUT), jnp.float32),
        compiler_params=pltpu.CompilerParams(
            dimension_semantics=("arbitrary",)),
        interpret=interpret,
    )(step_g, step_t, offsets, x_sorted, W1, b1[:, None, :], W2,
      b2[:, None, :])


_RANK_BLK = 128


def _routing(plane_idx, num_planes, num_tiles):
    n = plane_idx.shape[0]
    p32 = plane_idx.astype(jnp.int32)
    oh = (p32[:, None] == jnp.arange(num_planes, dtype=jnp.int32))
    ohf = oh.astype(jnp.float32)
    ohb = ohf.reshape(n // _RANK_BLK, _RANK_BLK, num_planes)
    tri = jnp.tril(jnp.ones((_RANK_BLK, _RANK_BLK), jnp.float32))
    rin = jnp.einsum("ij,bjp->bip", tri, ohb)
    bc = rin[:, -1, :]
    nblk = bc.shape[0]
    tri2 = jnp.tril(jnp.ones((nblk, nblk), jnp.float32), k=-1)
    bpre = tri2 @ bc
    rank = (rin + bpre[:, None, :]).reshape(n, num_planes)
    counts = bc.sum(axis=0).astype(jnp.int32)
    offsets = jnp.concatenate(
        [jnp.zeros((1,), jnp.int32),
         jnp.cumsum(counts).astype(jnp.int32)])
    rank_i = jnp.sum(jnp.where(oh, rank.astype(jnp.int32), 0), axis=1)
    off_i = jnp.sum(jnp.where(oh, offsets[None, :num_planes], 0), axis=1)
    pos = off_i + rank_i - 1
    t_start = offsets[:num_planes] // _B
    t_end = jnp.maximum(offsets[1:] - 1, 0) // _B
    nsteps = jnp.where(counts > 0, t_end - t_start + 1, 0).astype(jnp.int32)
    cum = jnp.cumsum(nsteps)
    first = cum - nsteps
    total = cum[-1]
    S = num_tiles + num_planes - 1
    s_idx = jnp.arange(S, dtype=jnp.int32)
    g_raw = jnp.searchsorted(cum, s_idx, side="right").astype(jnp.int32)
    real = s_idx < total
    g_last = jnp.searchsorted(cum, total - 1, side="right").astype(jnp.int32)
    g = jnp.where(real, jnp.minimum(g_raw, num_planes - 1), g_last)
    t = jnp.where(real, t_start[g] + s_idx - first[g],
                  num_tiles - 1).astype(jnp.int32)
    return pos, offsets, g, t


def kernel(x, plane_idx, W1, b1, W2, b2):
    N, D_IN = x.shape
    Lp = W1.shape[0]
    D_OUT = W2.shape[1]
    T = N // _B
    pos, offsets, step_g, step_t = _routing(plane_idx, Lp, T)
    x_sorted = _row_scatter_kernel(N, D_IN)(x, pos)
    out_sorted = _grouped_mlp(x_sorted, offsets, step_g, step_t,
                              W1, b1, W2, b2)
    return _row_gather_kernel(N, D_OUT)(out_sorted, pos)


# speedup vs baseline: 1.0136x; 1.0136x over previous
import functools

import jax
import jax.numpy as jnp
from jax import lax
from jax.experimental import pallas as pl
from jax.experimental.pallas import tpu as pltpu
from jax.experimental.pallas import tpu_sc as plsc

_B = 256

_NC = 2
_NS = 16
_NW = _NC * _NS
_CHUNK = 128


def _sc_mesh():
    return plsc.VectorSubcoreMesh(core_axis_name="c", subcore_axis_name="s",
                                  num_cores=_NC, num_subcores=_NS)


@functools.cache
def _row_gather_kernel(n_rows, dim):
    per_w = n_rows // _NW
    n_chunks = per_w // _CHUNK

    @functools.partial(
        pl.kernel,
        out_type=jax.ShapeDtypeStruct((n_rows, dim), jnp.float32),
        mesh=_sc_mesh(),
        scratch_types=[
            pltpu.VMEM((_CHUNK,), jnp.int32),
            pltpu.VMEM((_CHUNK, dim), jnp.float32),
            pltpu.SemaphoreType.DMA,
        ],
    )
    def gather_k(table_hbm, idx_hbm, out_hbm, idx_v, rows_v, sem):
        wid = lax.axis_index("s") * _NC + lax.axis_index("c")
        for j in range(n_chunks):
            base = wid * per_w + j * _CHUNK
            pltpu.sync_copy(idx_hbm.at[pl.ds(base, _CHUNK)], idx_v)
            pltpu.async_copy(table_hbm.at[idx_v], rows_v, sem).wait()
            pltpu.sync_copy(rows_v, out_hbm.at[pl.ds(base, _CHUNK)])

    return gather_k


@functools.cache
def _row_scatter_kernel(n_rows, dim):
    per_w = n_rows // _NW
    n_chunks = per_w // _CHUNK

    @functools.partial(
        pl.kernel,
        out_type=jax.ShapeDtypeStruct((n_rows, dim), jnp.float32),
        mesh=_sc_mesh(),
        scratch_types=[
            pltpu.VMEM((_CHUNK,), jnp.int32),
            pltpu.VMEM((_CHUNK, dim), jnp.float32),
            pltpu.SemaphoreType.DMA,
        ],
    )
    def scatter_k(rows_hbm, idx_hbm, out_hbm, idx_v, rows_v, sem):
        wid = lax.axis_index("s") * _NC + lax.axis_index("c")
        for j in range(n_chunks):
            base = wid * per_w + j * _CHUNK
            pltpu.sync_copy(idx_hbm.at[pl.ds(base, _CHUNK)], idx_v)
            pltpu.sync_copy(rows_hbm.at[pl.ds(base, _CHUNK)], rows_v)
            pltpu.async_copy(rows_v, out_hbm.at[idx_v], sem).wait()

    return scatter_k


def _mlp_step(sg_ref, st_ref, off_ref, x_ref, w1_ref, b1_ref, w2_ref, b2_ref,
              o_ref):
    s = pl.program_id(0)
    g = sg_ref[s]
    t = st_ref[s]
    lo = off_ref[g]
    hi = off_ref[g + 1]
    x = x_ref[...]
    h = lax.dot_general(x, w1_ref[0], (((1,), (1,)), ((), ())),
                        preferred_element_type=jnp.float32)
    h = h + b1_ref[0]
    h = 0.5 * h * (1.0 + lax.erf(h * (2.0 ** -0.5)))
    o = lax.dot_general(h, w2_ref[0], (((1,), (1,)), ((), ())),
                        preferred_element_type=jnp.float32)
    o = o + b2_ref[0]
    row = t * _B + lax.broadcasted_iota(jnp.int32, (_B, 1), 0)
    mask = (row >= lo) & (row < hi)
    o_ref[...] = jnp.where(mask, o, o_ref[...])


def _grouped_mlp(x_sorted, offsets, step_g, step_t, W1, b1, W2, b2, *,
                 interpret=False):
    N, D_IN = x_sorted.shape
    Lp, D_HID, _ = W1.shape
    D_OUT = W2.shape[1]
    S = step_g.shape[0]
    grid_spec = pltpu.PrefetchScalarGridSpec(
        num_scalar_prefetch=3,
        grid=(S,),
        in_specs=[
            pl.BlockSpec((_B, D_IN), lambda s, sg, st, off: (st[s], 0)),
            pl.BlockSpec((1, D_HID, D_IN), lambda s, sg, st, off: (sg[s], 0, 0)),
            pl.BlockSpec((1, 1, D_HID), lambda s, sg, st, off: (sg[s], 0, 0)),
            pl.BlockSpec((1, D_OUT, D_HID), lambda s, sg, st, off: (sg[s], 0, 0)),
            pl.BlockSpec((1, 1, D_OUT), lambda s, sg, st, off: (sg[s], 0, 0)),
        ],
        out_specs=pl.BlockSpec((_B, D_OUT), lambda s, sg, st, off: (st[s], 0)),
    )
    return pl.pallas_call(
        _mlp_step,
        grid_spec=grid_spec,
        out_shape=jax.ShapeDtypeStruct((N, D_OUT), jnp.float32),
        compiler_params=pltpu.CompilerParams(
            dimension_semantics=("arbitrary",)),
        interpret=interpret,
    )(step_g, step_t, offsets, x_sorted, W1, b1[:, None, :], W2,
      b2[:, None, :])


_RANK_BLK = 128


def _routing(plane_idx, num_planes, num_tiles):
    n = plane_idx.shape[0]
    p32 = plane_idx.astype(jnp.int32)
    oh = (p32[:, None] == jnp.arange(num_planes, dtype=jnp.int32))
    ohf = oh.astype(jnp.float32)
    ohb = ohf.reshape(n // _RANK_BLK, _RANK_BLK, num_planes)
    tri = jnp.tril(jnp.ones((_RANK_BLK, _RANK_BLK), jnp.float32))
    rin = jnp.einsum("ij,bjp->bip", tri, ohb)
    bc = rin[:, -1, :]
    nblk = bc.shape[0]
    tri2 = jnp.tril(jnp.ones((nblk, nblk), jnp.float32), k=-1)
    bpre = tri2 @ bc
    rank = (rin + bpre[:, None, :]).reshape(n, num_planes)
    counts = bc.sum(axis=0).astype(jnp.int32)
    offsets = jnp.concatenate(
        [jnp.zeros((1,), jnp.int32),
         jnp.cumsum(counts).astype(jnp.int32)])
    rank_i = jnp.sum(jnp.where(oh, rank.astype(jnp.int32), 0), axis=1)
    off_i = jnp.sum(jnp.where(oh, offsets[None, :num_planes], 0), axis=1)
    pos = off_i + rank_i - 1
    t_start = offsets[:num_planes] // _B
    t_end = jnp.maximum(offsets[1:] - 1, 0) // _B
    nsteps = jnp.where(counts > 0, t_end - t_start + 1, 0).astype(jnp.int32)
    cum = jnp.cumsum(nsteps)
    first = cum - nsteps
    total = cum[-1]
    S = num_tiles + num_planes - 1
    s_idx = jnp.arange(S, dtype=jnp.int32)
    g_raw = jnp.searchsorted(cum, s_idx, side="right").astype(jnp.int32)
    real = s_idx < total
    g_last = jnp.searchsorted(cum, total - 1, side="right").astype(jnp.int32)
    g = jnp.where(real, jnp.minimum(g_raw, num_planes - 1), g_last)
    t = jnp.where(real, t_start[g] + s_idx - first[g],
                  num_tiles - 1).astype(jnp.int32)
    return pos, offsets, g, t


def kernel(x, plane_idx, W1, b1, W2, b2):
    N, D_IN = x.shape
    Lp = W1.shape[0]
    D_OUT = W2.shape[1]
    T = N // _B
    pos, offsets, step_g, step_t = _routing(plane_idx, Lp, T)
    x_sorted = _row_scatter_kernel(N, D_IN)(x, pos)
    out_sorted = _grouped_mlp(x_sorted, offsets, step_g, step_t,
                              W1, b1, W2, b2)
    return _row_gather_kernel(N, D_OUT)(out_sorted, pos)
